# Initial kernel scaffold; baseline (speedup 1.0000x reference)
import jax
import jax.numpy as jnp
from jax.experimental import pallas as pl


def kernel(in_feats, in_coords, W, bias):
    raise NotImplementedError("write your pallas kernel here")


# speedup vs baseline: 25.8987x; 25.8987x over previous
import functools

import jax
import jax.numpy as jnp
from jax.experimental import pallas as pl
from jax.experimental.pallas import tpu as pltpu

N_TILES = 64
H = 8
CIN = 768
COUT = 768
B_TOKENS = 8192
A_SCALE = 16.0
A_BIAS = 0.5
R = 128
NBLK = 128
PB = NBLK * R


def _matmul_body(tile_map_ref, x_ref, w_ref, b_ref, o_ref):
    x = x_ref[...]
    w = w_ref[0]
    acc = jax.lax.dot_general(
        x, w, (((1,), (1,)), ((), ())), preferred_element_type=jnp.float32)
    acc = acc + b_ref[...]
    o_ref[...] = jnp.where(acc >= 0, acc, 0.2 * acc)


@functools.partial(jax.jit, static_argnames=())
def _grouped_matmul(tile_map, x_padded, W, bias2d):
    grid_spec = pltpu.PrefetchScalarGridSpec(
        num_scalar_prefetch=1,
        grid=(NBLK,),
        in_specs=[
            pl.BlockSpec((R, CIN), lambda i, tm: (i, 0)),
            pl.BlockSpec((1, COUT, CIN), lambda i, tm: (tm[i], 0, 0)),
            pl.BlockSpec((1, COUT), lambda i, tm: (0, 0)),
        ],
        out_specs=pl.BlockSpec((R, COUT), lambda i, tm: (i, 0)),
    )
    return pl.pallas_call(
        _matmul_body,
        grid_spec=grid_spec,
        out_shape=jax.ShapeDtypeStruct((PB, COUT), jnp.float32),
    )(tile_map, x_padded, W, bias2d)


def kernel(in_feats, in_coords, W, bias):
    aff = in_coords * A_SCALE + A_BIAS
    tx = jnp.floor(aff[:, 0]).astype(jnp.int32) % H
    ty = jnp.floor(aff[:, 1]).astype(jnp.int32) % H
    tile = H * tx + ty

    order = jnp.argsort(tile, stable=True)
    counts = jnp.bincount(tile, length=N_TILES)
    nb = (counts + R - 1) // R
    ends = jnp.cumsum(nb)
    bstart = ends - nb
    blk_ids = jnp.arange(NBLK)
    tile_map = jnp.minimum(
        jnp.searchsorted(ends, blk_ids, side="right"), N_TILES - 1
    ).astype(jnp.int32)

    seg_start = jnp.cumsum(counts) - counts
    ts = tile[order]
    ranks = jnp.arange(B_TOKENS) - seg_start[ts]
    pos = bstart[ts] * R + ranks
    gidx = jnp.zeros((PB,), jnp.int32).at[pos].set(order.astype(jnp.int32))
    pos_token = jnp.zeros((B_TOKENS,), jnp.int32).at[order].set(pos.astype(jnp.int32))

    x_padded = in_feats[gidx]
    out_padded = _grouped_matmul(tile_map, x_padded, W, bias.reshape(1, COUT))
    return out_padded[pos_token]


# speedup vs baseline: 54.8776x; 2.1189x over previous
import functools

import jax
import jax.numpy as jnp
from jax import lax
from jax.experimental import pallas as pl
from jax.experimental.pallas import tpu as pltpu
from jax.experimental.pallas import tpu_sc as plsc

N_TILES = 64
HGRID = 8
CIN = 768
COUT = 768
B_TOKENS = 8192
A_SCALE = 16.0
A_BIAS = 0.5

R = 128
NBLK = 128
PB = NBLK * R

NC = 2
NS = 16
NW = NC * NS
TOK_W = B_TOKENS // NW
NVEC = TOK_W // 16

_MESH = plsc.VectorSubcoreMesh(core_axis_name="c", subcore_axis_name="s",
                               num_cores=NC, num_subcores=NS)
_SC_PARAMS = pltpu.CompilerParams(needs_layout_passes=False)


def _worker_id():
    return lax.axis_index("s") * NC + lax.axis_index("c")


def _floor_i32(v):
    t = v.astype(jnp.int32)
    return jnp.where(t.astype(jnp.float32) > v, t - 1, t)


@functools.partial(
    pl.kernel,
    out_type=(
        jax.ShapeDtypeStruct((B_TOKENS,), jnp.int32),
        jax.ShapeDtypeStruct((B_TOKENS,), jnp.int32),
        jax.ShapeDtypeStruct((NW, N_TILES), jnp.int32),
    ),
    mesh=_MESH,
    compiler_params=_SC_PARAMS,
    scratch_types=(
        pltpu.VMEM((TOK_W,), jnp.float32),
        pltpu.VMEM((TOK_W,), jnp.float32),
        pltpu.VMEM((TOK_W,), jnp.int32),
        pltpu.VMEM((TOK_W,), jnp.int32),
        pltpu.VMEM((N_TILES,), jnp.int32),
    ),
)
def _route(coords_t, tiles_h, ranks_h, lcounts_h, cx_v, cy_v, tl_v, rk_v, cnt_v):
    w = _worker_id()
    base = w * TOK_W
    pltpu.sync_copy(coords_t.at[0, pl.ds(base, TOK_W)], cx_v)
    pltpu.sync_copy(coords_t.at[1, pl.ds(base, TOK_W)], cy_v)
    for c in range(N_TILES // 16):
        cnt_v[pl.ds(c * 16, 16)] = jnp.zeros((16,), jnp.int32)

    lane = lax.iota(jnp.int32, 16)

    def body(k, _):
        sl = pl.ds(k * 16, 16)
        mx = _floor_i32(cx_v[sl] * A_SCALE + A_BIAS) & (HGRID - 1)
        my = _floor_i32(cy_v[sl] * A_SCALE + A_BIAS) & (HGRID - 1)
        tile = mx * HGRID + my
        old = plsc.load_gather(cnt_v, [tile])
        rank = jnp.zeros((16,), jnp.int32)
        total = jnp.zeros((16,), jnp.int32)
        for l in range(16):
            tl = jnp.sum(jnp.where(lane == l, tile, 0))
            eq = tile == tl
            rank = rank + jnp.where(eq & (lane > l), 1, 0)
            total = total + jnp.where(eq, 1, 0)
        tl_v[sl] = tile
        rk_v[sl] = old + rank
        plsc.store_scatter(cnt_v, [tile], old + total)
        return 0

    lax.fori_loop(0, NVEC, body, 0)
    pltpu.sync_copy(tl_v, tiles_h.at[pl.ds(base, TOK_W)])
    pltpu.sync_copy(rk_v, ranks_h.at[pl.ds(base, TOK_W)])
    pltpu.sync_copy(cnt_v, lcounts_h.at[w])


@functools.partial(
    pl.kernel,
    out_type=(
        jax.ShapeDtypeStruct((PB, CIN), jnp.float32),
        jax.ShapeDtypeStruct((NW * 2, R), jnp.int32),
        jax.ShapeDtypeStruct((2, NBLK), jnp.int32),
    ),
    mesh=_MESH,
    compiler_params=_SC_PARAMS,
    scratch_types=(
        pltpu.VMEM((NW, N_TILES), jnp.int32),
        pltpu.VMEM((N_TILES,), jnp.int32),
        pltpu.VMEM((TOK_W,), jnp.int32),
        pltpu.VMEM((TOK_W,), jnp.int32),
        pltpu.VMEM((2, R), jnp.int32),
        pltpu.VMEM((NBLK,), jnp.int32),
        pltpu.VMEM((2, NBLK), jnp.int32),
        pltpu.VMEM((R, CIN), jnp.float32),
        pltpu.SemaphoreType.DMA,
    ),
)
def _dispatch(in_feats, tiles_h, ranks_h, lcounts_h,
              xpad_h, pos_h, meta_h,
              lc_v, base_v, tl_v, rk_v, pos_v, tm_v, meta_v, xb_v, sem):
    w = _worker_id()
    base = w * TOK_W
    pltpu.sync_copy(lcounts_h, lc_v)
    pltpu.sync_copy(tiles_h.at[pl.ds(base, TOK_W)], tl_v)
    pltpu.sync_copy(ranks_h.at[pl.ds(base, TOK_W)], rk_v)

    lane = lax.iota(jnp.int32, 16)
    for c in range(NBLK // 16):
        tm_v[pl.ds(c * 16, 16)] = jnp.zeros((16,), jnp.int32)
    used = jnp.int32(0)
    carry = jnp.int32(0)
    for c in range(N_TILES // 16):
        sl = pl.ds(c * 16, 16)

        def red(wp, acc):
            tot, pre = acc
            v = lc_v[wp, sl]
            tot = tot + v
            pre = pre + jnp.where(wp < w, v, 0)
            return (tot, pre)

        tot, pre = lax.fori_loop(
            0, NW, red, (jnp.zeros((16,), jnp.int32), jnp.zeros((16,), jnp.int32)))
        nb = (tot + (R - 1)) >> 7
        bstart = jnp.cumsum(nb) - nb + carry
        carry = carry + jnp.sum(nb)
        base_v[sl] = bstart * R + pre
        tvec = lane + c * 16
        plsc.store_scatter(tm_v, [jnp.minimum(bstart, NBLK - 1)],
                           tvec, mask=nb > 0)
        used = used + jnp.sum(nb)

    for k in range(NVEC):
        sl = pl.ds((k % (NVEC // 2)) * 16, 16)
        t = tl_v[pl.ds(k * 16, 16)]
        p = plsc.load_gather(base_v, [t]) + rk_v[pl.ds(k * 16, 16)]
        pos_v[k // (NVEC // 2), sl] = p

    for c in range(2):
        pltpu.sync_copy(in_feats.at[pl.ds(base + c * R, R)], xb_v)
        pltpu.async_copy(xb_v, xpad_h.at[pos_v.at[c]], sem).wait()
        pltpu.sync_copy(pos_v.at[c], pos_h.at[w * 2 + c])

    @pl.when(w == 0)
    def _():
        cmax = jnp.int32(0)
        for c in range(NBLK // 16):
            sl = pl.ds(c * 16, 16)
            v = jnp.maximum(plsc.cummax(tm_v[sl]), cmax)
            meta_v[0, sl] = v
            cmax = jnp.max(v)
            blk = lane + c * 16
            meta_v[1, sl] = jnp.where(blk < used, blk, NBLK - 1)
        pltpu.sync_copy(meta_v, meta_h)


def _matmul_body(meta_ref, x_ref, w_ref, b_ref, o_ref):
    x = x_ref[...]
    w = w_ref[0]
    acc = lax.dot_general(
        x, w, (((1,), (1,)), ((), ())), preferred_element_type=jnp.float32)
    acc = acc + b_ref[...]
    o_ref[...] = jnp.where(acc >= 0, acc, 0.2 * acc)


def _grouped_matmul(meta, x_padded, W, bias2d):
    grid_spec = pltpu.PrefetchScalarGridSpec(
        num_scalar_prefetch=1,
        grid=(NBLK,),
        in_specs=[
            pl.BlockSpec((R, CIN), lambda i, m: (m[1, i], 0)),
            pl.BlockSpec((1, COUT, CIN), lambda i, m: (m[0, i], 0, 0)),
            pl.BlockSpec((1, COUT), lambda i, m: (0, 0)),
        ],
        out_specs=pl.BlockSpec((R, COUT), lambda i, m: (m[1, i], 0)),
    )
    return pl.pallas_call(
        _matmul_body,
        grid_spec=grid_spec,
        out_shape=jax.ShapeDtypeStruct((PB, COUT), jnp.float32),
    )(meta, x_padded, W, bias2d)


@functools.partial(
    pl.kernel,
    out_type=jax.ShapeDtypeStruct((B_TOKENS, COUT), jnp.float32),
    mesh=_MESH,
    compiler_params=_SC_PARAMS,
    scratch_types=(
        pltpu.VMEM((2, R), jnp.int32),
        pltpu.VMEM((R, COUT), jnp.float32),
        pltpu.SemaphoreType.DMA,
    ),
)
def _unpermute(opad_h, pos_h, out_h, pos_v, ob_v, sem):
    w = _worker_id()
    pltpu.sync_copy(pos_h.at[pl.ds(w * 2, 2)], pos_v)
    for c in range(2):
        pltpu.async_copy(opad_h.at[pos_v.at[c]], ob_v, sem).wait()
        pltpu.sync_copy(ob_v, out_h.at[pl.ds(w * TOK_W + c * R, R)])


def kernel(in_feats, in_coords, W, bias):
    tiles_h, ranks_h, lcounts_h = _route(in_coords.T)
    x_padded, pos_h, meta = _dispatch(in_feats, tiles_h, ranks_h, lcounts_h)
    out_padded = _grouped_matmul(meta, x_padded, W, bias.reshape(1, COUT))
    return _unpermute(out_padded, pos_h)
